# Initial kernel scaffold; baseline (speedup 1.0000x reference)
import jax
import jax.numpy as jnp
from jax.experimental import pallas as pl


def kernel(x_post, x_user, x_entity, ei_publish, ei_repost, ei_contain, ei_interact, ei_follow, ei_similar, W_post, b_post, W_user, b_user, W_entity, b_entity, W_src, W_dst, att_src, att_dst, conv_bias, W_cls1, b_cls1, W_cls2, b_cls2):
    raise NotImplementedError("write your pallas kernel here")


# speedup vs baseline: 1.0511x; 1.0511x over previous
import functools

import jax
import jax.numpy as jnp
from jax.experimental import pallas as pl
from jax.experimental.pallas import tpu as pltpu

H = 2
C = 128


def _mm_body(x_ref, w_ref, b_ref, o_ref):
    o_ref[...] = (
        jnp.dot(x_ref[...], w_ref[...], preferred_element_type=jnp.float32)
        + b_ref[...]
    )


def _mm(x, w, b):
    M, K = x.shape
    N = w.shape[1]
    BM = 2048 if M > 2048 else M
    grid = (M + BM - 1) // BM
    return pl.pallas_call(
        _mm_body,
        grid=(grid,),
        in_specs=[
            pl.BlockSpec((BM, K), lambda i: (i, 0)),
            pl.BlockSpec((K, N), lambda i: (0, 0)),
            pl.BlockSpec((1, N), lambda i: (0, 0)),
        ],
        out_specs=pl.BlockSpec((BM, N), lambda i: (i, 0)),
        out_shape=jax.ShapeDtypeStruct((M, N), jnp.float32),
    )(x, w, b.reshape(1, N))


def _gat_edges(hs, al_s, al_d, ei, n_dst):
    src = ei[0]
    dst = ei[1]
    alpha = al_s[src] + al_d[dst]
    alpha = jax.nn.leaky_relu(alpha, negative_slope=0.2)
    ex = jnp.exp(alpha)
    denom = jax.ops.segment_sum(ex, dst, num_segments=n_dst)
    hs3 = hs.reshape(hs.shape[0], H, C)
    num = jax.ops.segment_sum(hs3[src] * ex[:, :, None], dst, num_segments=n_dst)
    agg = num / (denom[:, :, None] + 1e-16)
    return jnp.mean(agg, axis=1)


def kernel(x_post, x_user, x_entity, ei_publish, ei_repost, ei_contain,
           ei_interact, ei_follow, ei_similar, W_post, b_post, W_user, b_user,
           W_entity, b_entity, W_src, W_dst, att_src, att_dst, conv_bias,
           W_cls1, b_cls1, W_cls2, b_cls2):
    Ws4 = W_src.reshape(2, 6, C, H, C)
    Wd4 = W_dst.reshape(2, 6, C, H, C)
    V_src = jnp.einsum("lrkhc,lrhc->lrkh", Ws4, att_src)
    V_dst = jnp.einsum("lrkhc,lrhc->lrkh", Wd4, att_dst)

    hp = _mm(x_post, W_post, b_post)
    hu = _mm(x_user, W_user, b_user)
    he = _mm(x_entity, W_entity, b_entity)
    n_post, n_user, n_entity = hp.shape[0], hu.shape[0], he.shape[0]

    rel = {
        0: (ei_publish, "u", "p", n_post),
        1: (ei_repost, "u", "p", n_post),
        2: (ei_contain, "p", "e", n_entity),
        3: (ei_interact, "u", "u", n_user),
        4: (ei_follow, "u", "u", n_user),
        5: (ei_similar, "p", "p", n_post),
    }

    for l in range(2):
        xs = {"p": hp, "u": hu, "e": he}
        outs = {"p": None, "u": None, "e": None}
        for r in range(6):
            ei, s, d, nd = rel[r]
            Wcat = jnp.concatenate(
                [W_src[l, r], V_src[l, r], jnp.zeros((C, 128 - H), jnp.float32)],
                axis=1,
            )
            big = _mm(xs[s], Wcat, jnp.zeros((Wcat.shape[1],), jnp.float32))
            hs = big[:, : H * C]
            al_s = big[:, H * C : H * C + H]
            Vd = jnp.concatenate(
                [V_dst[l, r], jnp.zeros((C, 128 - H), jnp.float32)], axis=1
            )
            al_d = _mm(xs[d], Vd, jnp.zeros((128,), jnp.float32))[:, :H]
            agg = _gat_edges(hs, al_s, al_d, ei, nd) + conv_bias[l, r]
            outs[d] = agg if outs[d] is None else outs[d] + agg
        hp = jax.nn.relu(outs["p"])
        hu = jax.nn.relu(outs["u"])
        he = jax.nn.relu(outs["e"])

    h = jax.nn.relu(_mm(hp, W_cls1, b_cls1))
    W2 = jnp.concatenate([W_cls2, jnp.zeros((W_cls2.shape[0], 127), jnp.float32)], axis=1)
    b2 = jnp.concatenate([b_cls2, jnp.zeros((127,), jnp.float32)])
    out = _mm(h, W2, b2)[:, 0]
    return out
